# Initial kernel scaffold; baseline (speedup 1.0000x reference)
import jax
import jax.numpy as jnp
from jax.experimental import pallas as pl


def kernel(x, edge_index, edge_mask, y_true, W, w_out):
    raise NotImplementedError("write your pallas kernel here")


# speedup vs baseline: 5.2212x; 5.2212x over previous
import functools

import jax
import jax.numpy as jnp
from jax import lax
from jax.experimental import pallas as pl
from jax.experimental.pallas import tpu as pltpu
from jax.experimental.pallas import tpu_sc as plsc

_N = 10000
_E = 320000
_D = 128
_EPS = 1e-15
_ENT_EDGE_SIZE = 0.005
_ENT_EDGE_ENTROPY = 1.0

_NC = 2
_NS = 16
_NW = _NC * _NS
_CHUNK = 128
_NCH = _E // _CHUNK
_ROWS_PER_TILE = _N // _NS


def _sigmoid(z):
    return 1.0 / (1.0 + jnp.exp(-z))


def _build_sc_agg():
    mesh = plsc.VectorSubcoreMesh(
        core_axis_name="c", subcore_axis_name="s",
        num_cores=_NC, num_subcores=_NS)

    @functools.partial(
        pl.kernel,
        mesh=mesh,
        out_type=jax.ShapeDtypeStruct((_NC, _NS, _ROWS_PER_TILE, _D),
                                      jnp.float32),
        scratch_types=[
            pltpu.VMEM((_CHUNK,), jnp.int32),
            pltpu.VMEM((1, _CHUNK), jnp.int32),
            pltpu.VMEM((_CHUNK,), jnp.float32),
            pltpu.VMEM((_CHUNK, _D), jnp.float32),
            pltpu.SemaphoreType.DMA,
            pltpu.VMEM_SHARED((_N, _D), jnp.float32),
        ],
    )
    def sc_agg(x_hbm, ei_hbm, mask_hbm, out_hbm,
               src_v, dst_v, m_v, rows_v, sem, agg_sh):
        cid = lax.axis_index("c")
        sid = lax.axis_index("s")
        wid = cid * _NS + sid

        def zrow(i, carry):
            for f in range(_D // 16):
                rows_v[i, pl.ds(f * 16, 16)] = jnp.zeros((16,), jnp.float32)
            return carry
        lax.fori_loop(0, _CHUNK, zrow, 0)
        for i in range(5):
            n = _CHUNK if i < 4 else _ROWS_PER_TILE - 4 * _CHUNK
            pltpu.sync_copy(
                rows_v.at[pl.ds(0, n)],
                agg_sh.at[pl.ds(sid * _ROWS_PER_TILE + i * _CHUNK, n)])
        plsc.subcore_barrier()

        n_my = jnp.where(wid < _NCH % _NW, _NCH // _NW + 1, _NCH // _NW)

        def chunk_body(j, carry):
            base = (wid + j * _NW) * _CHUNK
            pltpu.sync_copy(ei_hbm.at[0, pl.ds(base, _CHUNK)], src_v)
            pltpu.sync_copy(ei_hbm.at[1, pl.ds(base, _CHUNK)], dst_v.at[0])
            pltpu.sync_copy(mask_hbm.at[pl.ds(base, _CHUNK)], m_v)
            gat = pltpu.async_copy(x_hbm.at[src_v], rows_v, sem)
            for f in range(_CHUNK // 16):
                sl = pl.ds(f * 16, 16)
                m_v[sl] = _sigmoid(m_v[sl])
            gat.wait()

            def group_body(g, carry2):
                m_seg = m_v[pl.ds(g * 16, 16)]
                for i in range(16):
                    mb = jnp.full((16,), m_seg[i])
                    e = g * 16 + i
                    for f in range(_D // 16):
                        sl = pl.ds(f * 16, 16)
                        rows_v[e, sl] = rows_v[e, sl] * mb
                return carry2
            lax.fori_loop(0, _CHUNK // 16, group_body, 0)

            pltpu.sync_copy(rows_v, agg_sh.at[dst_v.at[0]], add=True)
            return carry
        lax.fori_loop(0, n_my, chunk_body, 0)

        plsc.subcore_barrier()

        pltpu.sync_copy(
            agg_sh.at[pl.ds(sid * _ROWS_PER_TILE, _ROWS_PER_TILE)],
            out_hbm.at[cid, sid])

    return sc_agg


_sc_agg = _build_sc_agg()


def _tc_head_body(aggs_ref, mask_ref, y_ref, w_ref, wout_ref, out_ref):
    agg = aggs_ref[0] + aggs_ref[1]
    h = jnp.maximum(
        jnp.dot(agg, w_ref[...], preferred_element_type=jnp.float32), 0.0)
    lt = lax.dot_general(wout_ref[...], h, (((0,), (1,)), ((), ())),
                         preferred_element_type=jnp.float32)
    y = y_ref[...]
    bce = (jnp.maximum(lt, 0.0) - lt * y
           + jnp.log(1.0 + jnp.exp(-jnp.abs(lt))))
    m = _sigmoid(mask_ref[...])
    ent = (-m * jnp.log(m + _EPS)
           - (1.0 - m) * jnp.log(1.0 - m + _EPS))
    loss = (jnp.sum(bce) / _N
            + _ENT_EDGE_SIZE * jnp.sum(m)
            + _ENT_EDGE_ENTROPY * (jnp.sum(ent) / _E))
    out_ref[...] = jnp.full((1, 1), loss, jnp.float32)


def _tc_head(aggs, mask2d, y_row, W, w_col):
    return pl.pallas_call(
        _tc_head_body,
        out_shape=jax.ShapeDtypeStruct((1, 1), jnp.float32),
    )(aggs, mask2d, y_row, W, w_col)


def kernel(x, edge_index, edge_mask, y_true, W, w_out):
    aggs = _sc_agg(x, edge_index, edge_mask).reshape(_NC, _N, _D)
    loss = _tc_head(aggs, edge_mask.reshape(_E // _D, _D),
                    y_true.reshape(1, _N), W, w_out.reshape(_D, 1))
    return loss.reshape(())


# speedup vs baseline: 11.3317x; 2.1703x over previous
import functools

import jax
import jax.numpy as jnp
from jax import lax
from jax.experimental import pallas as pl
from jax.experimental.pallas import tpu as pltpu
from jax.experimental.pallas import tpu_sc as plsc

_N = 10000
_E = 320000
_D = 128
_EPS = 1e-15
_ENT_EDGE_SIZE = 0.005
_ENT_EDGE_ENTROPY = 1.0

_NC = 2
_NS = 16
_NW = _NC * _NS
_CHUNK = 128
_NCH = _E // _CHUNK
_NMAIN = _NCH // _NW
_G = 6
_NG = _NMAIN // _G
_ROWS_PER_TILE = _N // _NS


def _sigmoid(z):
    return 1.0 / (1.0 + jnp.exp(-z))


def _build_sc_agg():
    mesh = plsc.VectorSubcoreMesh(
        core_axis_name="c", subcore_axis_name="s",
        num_cores=_NC, num_subcores=_NS)

    @functools.partial(
        pl.kernel,
        mesh=mesh,
        out_type=jax.ShapeDtypeStruct((_NC, _NS, _ROWS_PER_TILE, _D),
                                      jnp.float32),
        scratch_types=[
            pltpu.VMEM((2, _CHUNK), jnp.int32),
            pltpu.VMEM((2, _G * _CHUNK), jnp.int32),
            pltpu.VMEM((2, _G * _CHUNK), jnp.float32),
            pltpu.VMEM((2, _CHUNK, _D), jnp.float32),
            pltpu.VMEM((1, _CHUNK), jnp.int32),
            pltpu.SemaphoreType.DMA,
            pltpu.SemaphoreType.DMA,
            pltpu.SemaphoreType.DMA,
            pltpu.SemaphoreType.DMA,
            pltpu.SemaphoreType.DMA,
            pltpu.VMEM_SHARED((_N, _D), jnp.float32),
        ],
    )
    def sc_agg(x_hbm, ei_hbm, mask_hbm, out_hbm,
               dst_v, src_blk, m_blk, rows_v, dstx_v,
               sem_ib, sem_g0, sem_g1, sem_s0, sem_s1, agg_sh):
        cid = lax.axis_index("c")
        sid = lax.axis_index("s")
        wid = cid * _NS + sid
        cstart = wid * _NMAIN
        sem_g = (sem_g0, sem_g1)
        sem_s = (sem_s0, sem_s1)

        def fetch_blk(g, b2):
            base = (cstart + g * _G) * _CHUNK
            pltpu.async_copy(
                ei_hbm.at[0, pl.ds(base, _G * _CHUNK)],
                src_blk.at[b2], sem_ib)
            pltpu.async_copy(
                mask_hbm.at[pl.ds(base, _G * _CHUNK)],
                m_blk.at[b2], sem_ib)

        def wait_blk(g, b2):
            base = (cstart + g * _G) * _CHUNK
            pltpu.make_async_copy(
                ei_hbm.at[0, pl.ds(base, _G * _CHUNK)],
                src_blk.at[b2], sem_ib).wait()
            pltpu.make_async_copy(
                mask_hbm.at[pl.ds(base, _G * _CHUNK)],
                m_blk.at[b2], sem_ib).wait()

        def issue_gather(b, b2, i, k):
            pltpu.async_copy(
                x_hbm.at[src_blk.at[b2, pl.ds(i * _CHUNK, _CHUNK)]],
                rows_v.at[b], sem_g[b])
            pltpu.async_copy(
                ei_hbm.at[1, pl.ds((cstart + k) * _CHUNK, _CHUNK)],
                dst_v.at[b], sem_g[b])

        def wait_gather(b, b2, i, k):
            pltpu.make_async_copy(
                x_hbm.at[src_blk.at[b2, pl.ds(i * _CHUNK, _CHUNK)]],
                rows_v.at[b], sem_g[b]).wait()
            pltpu.make_async_copy(
                ei_hbm.at[1, pl.ds((cstart + k) * _CHUNK, _CHUNK)],
                dst_v.at[b], sem_g[b]).wait()

        def issue_scatter(b):
            pltpu.async_copy(
                rows_v.at[b], agg_sh.at[dst_v.at[b]], sem_s[b], add=True)

        def wait_scatter(b):
            pltpu.make_async_copy(
                rows_v.at[b], agg_sh.at[dst_v.at[b]], sem_s[b]).wait()

        def scale_rows(b, m_b2, i):
            def g16(g8, carry):
                m_seg = m_blk[m_b2, pl.ds(i * _CHUNK + g8 * 16, 16)]
                for lane in range(16):
                    mb = jnp.full((16,), m_seg[lane])
                    e = g8 * 16 + lane
                    for f in range(_D // 16):
                        sl = pl.ds(f * 16, 16)
                        rows_v[b, e, sl] = rows_v[b, e, sl] * mb
                return carry
            lax.fori_loop(0, _CHUNK // 16, g16, 0)

        def tail(b, m_b2, i, k):
            wait_gather(b, m_b2, i, k)
            scale_rows(b, m_b2, i)
            issue_scatter(b)

        fetch_blk(0, 0)

        def zrow(i, carry):
            for f in range(_D // 16):
                rows_v[0, i, pl.ds(f * 16, 16)] = jnp.zeros((16,), jnp.float32)
            return carry
        lax.fori_loop(0, _CHUNK, zrow, 0)
        for i in range(5):
            n = _CHUNK if i < 4 else _ROWS_PER_TILE - 4 * _CHUNK
            pltpu.sync_copy(
                rows_v.at[0, pl.ds(0, n)],
                agg_sh.at[pl.ds(sid * _ROWS_PER_TILE + i * _CHUNK, n)])
        plsc.subcore_barrier()

        def group_body(g, carry):
            b2 = g % 2
            pb2 = (g + 1) % 2
            k0 = 6 * g
            wait_blk(g, b2)

            @pl.when(g >= 1)
            def _():
                wait_scatter(0)
            issue_gather(0, b2, 0, k0)

            @pl.when(g >= 1)
            def _():
                tail(1, pb2, 5, k0 - 1)

            for j in range(_G * _CHUNK // 16):
                sl = pl.ds(j * 16, 16)
                m_blk[b2, sl] = _sigmoid(m_blk[b2, sl])

            @pl.when(g >= 1)
            def _():
                wait_scatter(1)
            issue_gather(1, b2, 1, k0 + 1)
            tail(0, b2, 0, k0)

            @pl.when(g < _NG - 1)
            def _():
                fetch_blk(g + 1, pb2)

            wait_scatter(0)
            issue_gather(0, b2, 2, k0 + 2)
            tail(1, b2, 1, k0 + 1)

            wait_scatter(1)
            issue_gather(1, b2, 3, k0 + 3)
            tail(0, b2, 2, k0 + 2)

            wait_scatter(0)
            issue_gather(0, b2, 4, k0 + 4)
            tail(1, b2, 3, k0 + 3)

            wait_scatter(1)
            issue_gather(1, b2, 5, k0 + 5)
            tail(0, b2, 4, k0 + 4)
            return carry
        lax.fori_loop(0, _NG, group_body, 0)

        last_b2 = (_NG - 1) % 2
        tail(1, last_b2, 5, _NMAIN - 1)
        wait_scatter(0)
        wait_scatter(1)

        @pl.when(sid < 2)
        def _():
            exb = (_NW * _NMAIN + sid * 2 + cid) * _CHUNK
            pltpu.sync_copy(ei_hbm.at[0, pl.ds(exb, _CHUNK)],
                            src_blk.at[0, pl.ds(0, _CHUNK)])
            pltpu.sync_copy(ei_hbm.at[1, pl.ds(exb, _CHUNK)], dstx_v.at[0])
            pltpu.sync_copy(mask_hbm.at[pl.ds(exb, _CHUNK)],
                            m_blk.at[0, pl.ds(0, _CHUNK)])
            for f in range(_CHUNK // 16):
                sl = pl.ds(f * 16, 16)
                m_blk[0, sl] = _sigmoid(m_blk[0, sl])
            pltpu.async_copy(
                x_hbm.at[src_blk.at[0, pl.ds(0, _CHUNK)]],
                rows_v.at[0], sem_g[0])
            pltpu.make_async_copy(
                x_hbm.at[src_blk.at[0, pl.ds(0, _CHUNK)]],
                rows_v.at[0], sem_g[0]).wait()
            scale_rows(0, 0, 0)
            pltpu.sync_copy(rows_v.at[0], agg_sh.at[dstx_v.at[0]], add=True)

        plsc.subcore_barrier()

        pltpu.sync_copy(
            agg_sh.at[pl.ds(sid * _ROWS_PER_TILE, _ROWS_PER_TILE)],
            out_hbm.at[cid, sid])

    return sc_agg


_sc_agg = _build_sc_agg()


def _tc_head_body(aggs_ref, mask_ref, y_ref, w_ref, wout_ref, out_ref):
    agg = aggs_ref[0] + aggs_ref[1]
    h = jnp.maximum(
        jnp.dot(agg, w_ref[...], preferred_element_type=jnp.float32), 0.0)
    lt = lax.dot_general(wout_ref[...], h, (((0,), (1,)), ((), ())),
                         preferred_element_type=jnp.float32)
    y = y_ref[...]
    bce = (jnp.maximum(lt, 0.0) - lt * y
           + jnp.log(1.0 + jnp.exp(-jnp.abs(lt))))
    m = _sigmoid(mask_ref[...])
    ent = (-m * jnp.log(m + _EPS)
           - (1.0 - m) * jnp.log(1.0 - m + _EPS))
    loss = (jnp.sum(bce) / _N
            + _ENT_EDGE_SIZE * jnp.sum(m)
            + _ENT_EDGE_ENTROPY * (jnp.sum(ent) / _E))
    out_ref[...] = jnp.full((1, 1), loss, jnp.float32)


def _tc_head(aggs, mask2d, y_row, W, w_col):
    return pl.pallas_call(
        _tc_head_body,
        out_shape=jax.ShapeDtypeStruct((1, 1), jnp.float32),
    )(aggs, mask2d, y_row, W, w_col)


def kernel(x, edge_index, edge_mask, y_true, W, w_out):
    aggs = _sc_agg(x, edge_index, edge_mask).reshape(_NC, _N, _D)
    loss = _tc_head(aggs, edge_mask.reshape(_E // _D, _D),
                    y_true.reshape(1, _N), W, w_out.reshape(_D, 1))
    return loss.reshape(())


# speedup vs baseline: 11.9817x; 1.0574x over previous
import functools

import jax
import jax.numpy as jnp
from jax import lax
from jax.experimental import pallas as pl
from jax.experimental.pallas import tpu as pltpu
from jax.experimental.pallas import tpu_sc as plsc

_N = 10000
_E = 320000
_D = 128
_EPS = 1e-15
_ENT_EDGE_SIZE = 0.005
_ENT_EDGE_ENTROPY = 1.0

_NC = 2
_NS = 16
_NW = _NC * _NS
_CHUNK = 128
_NCH = _E // _CHUNK
_NMAIN = _NCH // _NW
_G = 6
_NG = _NMAIN // _G
_ROWS_PER_TILE = _N // _NS
_RB = 624


def _sigmoid(z):
    return 1.0 / (1.0 + jnp.exp(-z))


def _build_sc_agg():
    mesh = plsc.VectorSubcoreMesh(
        core_axis_name="c", subcore_axis_name="s",
        num_cores=_NC, num_subcores=_NS)

    @functools.partial(
        pl.kernel,
        mesh=mesh,
        out_type=jax.ShapeDtypeStruct((_NC, _N, _D), jnp.float32),
        scratch_types=[
            pltpu.VMEM((2, _CHUNK), jnp.int32),
            pltpu.VMEM((2, _G * _CHUNK), jnp.int32),
            pltpu.VMEM((2, _G * _CHUNK), jnp.float32),
            pltpu.VMEM((2, _CHUNK, _D), jnp.float32),
            pltpu.VMEM((1, _CHUNK), jnp.int32),
            pltpu.SemaphoreType.DMA,
            pltpu.SemaphoreType.DMA,
            pltpu.SemaphoreType.DMA,
            pltpu.SemaphoreType.DMA,
            pltpu.SemaphoreType.DMA,
            pltpu.VMEM_SHARED((_N, _D), jnp.float32),
        ],
    )
    def sc_agg(x_hbm, ei_hbm, mask_hbm, out_hbm,
               dst_v, src_blk, m_blk, rows_v, dstx_v,
               sem_ib, sem_g0, sem_g1, sem_s0, sem_s1, agg_sh):
        cid = lax.axis_index("c")
        sid = lax.axis_index("s")
        wid = cid * _NS + sid
        cstart = wid * _NMAIN
        sem_g = (sem_g0, sem_g1)
        sem_s = (sem_s0, sem_s1)

        def fetch_blk(g, b2):
            base = (cstart + g * _G) * _CHUNK
            pltpu.async_copy(
                ei_hbm.at[0, pl.ds(base, _G * _CHUNK)],
                src_blk.at[b2], sem_ib)
            pltpu.async_copy(
                mask_hbm.at[pl.ds(base, _G * _CHUNK)],
                m_blk.at[b2], sem_ib)

        def wait_blk(g, b2):
            base = (cstart + g * _G) * _CHUNK
            pltpu.make_async_copy(
                ei_hbm.at[0, pl.ds(base, _G * _CHUNK)],
                src_blk.at[b2], sem_ib).wait()
            pltpu.make_async_copy(
                mask_hbm.at[pl.ds(base, _G * _CHUNK)],
                m_blk.at[b2], sem_ib).wait()

        def issue_gather(b, b2, i, k):
            pltpu.async_copy(
                x_hbm.at[src_blk.at[b2, pl.ds(i * _CHUNK, _CHUNK)]],
                rows_v.at[b], sem_g[b])
            pltpu.async_copy(
                ei_hbm.at[1, pl.ds((cstart + k) * _CHUNK, _CHUNK)],
                dst_v.at[b], sem_g[b])

        def wait_gather(b, b2, i, k):
            pltpu.make_async_copy(
                x_hbm.at[src_blk.at[b2, pl.ds(i * _CHUNK, _CHUNK)]],
                rows_v.at[b], sem_g[b]).wait()
            pltpu.make_async_copy(
                ei_hbm.at[1, pl.ds((cstart + k) * _CHUNK, _CHUNK)],
                dst_v.at[b], sem_g[b]).wait()

        def issue_scatter(b):
            pltpu.async_copy(
                rows_v.at[b], agg_sh.at[dst_v.at[b]], sem_s[b], add=True)

        def wait_scatter(b):
            pltpu.make_async_copy(
                rows_v.at[b], agg_sh.at[dst_v.at[b]], sem_s[b]).wait()

        def scale_rows(b, m_b2, i):
            def g16(g8, carry):
                m_seg = m_blk[m_b2, pl.ds(i * _CHUNK + g8 * 16, 16)]
                for lane in range(16):
                    mb = jnp.full((16,), m_seg[lane])
                    e = g8 * 16 + lane
                    for f in range(_D // 16):
                        sl = pl.ds(f * 16, 16)
                        rows_v[b, e, sl] = rows_v[b, e, sl] * mb
                return carry
            lax.fori_loop(0, _CHUNK // 16, g16, 0)

        def tail(b, m_b2, i, k):
            wait_gather(b, m_b2, i, k)
            scale_rows(b, m_b2, i)
            issue_scatter(b)

        fetch_blk(0, 0)

        def zrow(i, carry):
            for f in range(_D // 16):
                rows_v[0, i, pl.ds(f * 16, 16)] = jnp.zeros((16,), jnp.float32)
            return carry
        lax.fori_loop(0, _CHUNK, zrow, 0)
        for i in range(5):
            n = _CHUNK if i < 4 else _RB - 4 * _CHUNK
            pltpu.sync_copy(
                rows_v.at[0, pl.ds(0, n)],
                agg_sh.at[pl.ds(sid * _RB + i * _CHUNK, n)])

        @pl.when(sid == 0)
        def _():
            pltpu.sync_copy(
                rows_v.at[0, pl.ds(0, _N - _NS * _RB)],
                agg_sh.at[pl.ds(_NS * _RB, _N - _NS * _RB)])
        plsc.subcore_barrier()

        def group_body(g, carry):
            b2 = g % 2
            pb2 = (g + 1) % 2
            k0 = 6 * g
            wait_blk(g, b2)

            @pl.when(g >= 1)
            def _():
                wait_scatter(0)
            issue_gather(0, b2, 0, k0)

            @pl.when(g >= 1)
            def _():
                tail(1, pb2, 5, k0 - 1)

            for j in range(_G * _CHUNK // 16):
                sl = pl.ds(j * 16, 16)
                m_blk[b2, sl] = _sigmoid(m_blk[b2, sl])

            @pl.when(g >= 1)
            def _():
                wait_scatter(1)
            issue_gather(1, b2, 1, k0 + 1)
            tail(0, b2, 0, k0)

            @pl.when(g < _NG - 1)
            def _():
                fetch_blk(g + 1, pb2)

            wait_scatter(0)
            issue_gather(0, b2, 2, k0 + 2)
            tail(1, b2, 1, k0 + 1)

            wait_scatter(1)
            issue_gather(1, b2, 3, k0 + 3)
            tail(0, b2, 2, k0 + 2)

            wait_scatter(0)
            issue_gather(0, b2, 4, k0 + 4)
            tail(1, b2, 3, k0 + 3)

            wait_scatter(1)
            issue_gather(1, b2, 5, k0 + 5)
            tail(0, b2, 4, k0 + 4)
            return carry
        lax.fori_loop(0, _NG, group_body, 0)

        last_b2 = (_NG - 1) % 2
        tail(1, last_b2, 5, _NMAIN - 1)
        wait_scatter(0)
        wait_scatter(1)

        @pl.when(sid < 2)
        def _():
            exb = (_NW * _NMAIN + sid * 2 + cid) * _CHUNK
            pltpu.sync_copy(ei_hbm.at[0, pl.ds(exb, _CHUNK)],
                            src_blk.at[0, pl.ds(0, _CHUNK)])
            pltpu.sync_copy(ei_hbm.at[1, pl.ds(exb, _CHUNK)], dstx_v.at[0])
            pltpu.sync_copy(mask_hbm.at[pl.ds(exb, _CHUNK)],
                            m_blk.at[0, pl.ds(0, _CHUNK)])
            for f in range(_CHUNK // 16):
                sl = pl.ds(f * 16, 16)
                m_blk[0, sl] = _sigmoid(m_blk[0, sl])
            pltpu.async_copy(
                x_hbm.at[src_blk.at[0, pl.ds(0, _CHUNK)]],
                rows_v.at[0], sem_g0)
            pltpu.make_async_copy(
                x_hbm.at[src_blk.at[0, pl.ds(0, _CHUNK)]],
                rows_v.at[0], sem_g0).wait()
            scale_rows(0, 0, 0)
            pltpu.sync_copy(rows_v.at[0], agg_sh.at[dstx_v.at[0]], add=True)

        plsc.subcore_barrier()

        pltpu.sync_copy(
            agg_sh.at[pl.ds(sid * _RB, _RB)],
            out_hbm.at[cid, pl.ds(sid * _RB, _RB)])

        @pl.when(sid == 0)
        def _():
            pltpu.sync_copy(
                agg_sh.at[pl.ds(_NS * _RB, _N - _NS * _RB)],
                out_hbm.at[cid, pl.ds(_NS * _RB, _N - _NS * _RB)])

    return sc_agg


_sc_agg = _build_sc_agg()


def _tc_head_body(aggs_ref, mask_ref, y_ref, w_ref, wout_ref, out_ref):
    agg = aggs_ref[0] + aggs_ref[1]
    h = jnp.maximum(
        jnp.dot(agg, w_ref[...], preferred_element_type=jnp.float32), 0.0)
    lt = lax.dot_general(wout_ref[...], h, (((0,), (1,)), ((), ())),
                         preferred_element_type=jnp.float32)
    y = y_ref[...]
    bce = (jnp.maximum(lt, 0.0) - lt * y
           + jnp.log(1.0 + jnp.exp(-jnp.abs(lt))))
    m = _sigmoid(mask_ref[...])
    ent = (-m * jnp.log(m + _EPS)
           - (1.0 - m) * jnp.log(1.0 - m + _EPS))
    loss = (jnp.sum(bce) / _N
            + _ENT_EDGE_SIZE * jnp.sum(m)
            + _ENT_EDGE_ENTROPY * (jnp.sum(ent) / _E))
    out_ref[...] = jnp.full((1, 1), loss, jnp.float32)


def _tc_head(aggs, mask2d, y_row, W, w_col):
    return pl.pallas_call(
        _tc_head_body,
        out_shape=jax.ShapeDtypeStruct((1, 1), jnp.float32),
    )(aggs, mask2d, y_row, W, w_col)


def kernel(x, edge_index, edge_mask, y_true, W, w_out):
    aggs = _sc_agg(x, edge_index, edge_mask)
    loss = _tc_head(aggs, edge_mask.reshape(_E // _D, _D),
                    y_true.reshape(1, _N), W, w_out.reshape(_D, 1))
    return loss.reshape(())
